# Initial kernel scaffold; baseline (speedup 1.0000x reference)
import jax
import jax.numpy as jnp
from jax.experimental import pallas as pl


def kernel(x, edge_index, W1, b1, W2, b2, W3, b3):
    raise NotImplementedError("write your pallas kernel here")


# speedup vs baseline: 17.4508x; 17.4508x over previous
import functools

import jax
import jax.numpy as jnp
from jax import lax
from jax.experimental import pallas as pl
from jax.experimental.pallas import tpu as pltpu
from jax.experimental.pallas import tpu_sc as plsc

N = 10000
E = 320000
D = 128
NPAD = 10240
NW = 32
EPW = E // NW
CD = 2000
CA = 200
RB = 2000

_mesh = functools.partial(
    plsc.VectorSubcoreMesh, core_axis_name="c", subcore_axis_name="s")


@functools.partial(
    pl.kernel,
    out_type=jax.ShapeDtypeStruct((2 * NPAD,), jnp.float32),
    mesh=_mesh(),
    scratch_types=[
        pltpu.VMEM((CD,), jnp.int32),
        pltpu.VMEM((CD,), jnp.float32),
        pltpu.VMEM((640,), jnp.float32),
        pltpu.VMEM_SHARED((NPAD,), jnp.float32),
    ],
)
def _sc_degree(dst_hbm, out_hbm, idx_v, ones_v, zero_v, deg_sh):
    c = lax.axis_index("c")
    s = lax.axis_index("s")
    wid = c * 16 + s

    def fill_ones(i, carry):
        ones_v[pl.ds(i * 16, 16)] = jnp.full((16,), 1.0, jnp.float32)
        return carry

    lax.fori_loop(0, CD // 16, fill_ones, 0)

    def fill_zero(i, carry):
        zero_v[pl.ds(i * 16, 16)] = jnp.zeros((16,), jnp.float32)
        return carry

    lax.fori_loop(0, 640 // 16, fill_zero, 0)

    pltpu.sync_copy(zero_v, deg_sh.at[pl.ds(s * 640, 640)])
    plsc.subcore_barrier()

    def chunk(k, carry):
        base = wid * EPW + k * CD
        pltpu.sync_copy(dst_hbm.at[pl.ds(base, CD)], idx_v)
        pltpu.sync_copy(ones_v, deg_sh.at[idx_v], add=True)
        return carry

    lax.fori_loop(0, EPW // CD, chunk, 0)
    plsc.subcore_barrier()

    pltpu.sync_copy(deg_sh.at[pl.ds(s * 640, 640)],
                    out_hbm.at[pl.ds(c * NPAD + s * 640, 640)])


@functools.partial(
    pl.kernel,
    out_type=(jax.ShapeDtypeStruct((NPAD, D), jnp.float32),
              jax.ShapeDtypeStruct((NPAD, D), jnp.float32)),
    mesh=_mesh(),
    scratch_types=[
        pltpu.VMEM((CA,), jnp.int32),
        pltpu.VMEM((CA,), jnp.int32),
        pltpu.VMEM((CA, D), jnp.float32),
        pltpu.VMEM_SHARED((NPAD, D), jnp.float32),
        pltpu.SemaphoreType.DMA,
    ],
)
def _sc_aggregate(g_hbm, src_hbm, dst_hbm, out0_hbm, out1_hbm,
                  src_v, dst_v, rows_v, acc_sh, sem):
    c = lax.axis_index("c")
    s = lax.axis_index("s")
    wid = c * 16 + s

    def fill_zero(i, carry):
        rows_v[i // 8, pl.ds((i % 8) * 16, 16)] = jnp.zeros((16,), jnp.float32)
        return carry

    lax.fori_loop(0, CA * (D // 16), fill_zero, 0)

    row0 = s * 640
    for off in (0, 200, 400):
        pltpu.sync_copy(rows_v, acc_sh.at[pl.ds(row0 + off, CA)])
    pltpu.sync_copy(rows_v.at[pl.ds(0, 40)], acc_sh.at[pl.ds(row0 + 600, 40)])
    plsc.subcore_barrier()

    def chunk(k, carry):
        base = wid * EPW + k * CA
        pltpu.sync_copy(src_hbm.at[pl.ds(base, CA)], src_v)
        pltpu.sync_copy(dst_hbm.at[pl.ds(base, CA)], dst_v)
        pltpu.async_copy(g_hbm.at[src_v], rows_v, sem).wait()
        pltpu.sync_copy(rows_v, acc_sh.at[dst_v], add=True)
        return carry

    lax.fori_loop(0, EPW // CA, chunk, 0)
    plsc.subcore_barrier()

    @pl.when(c == 0)
    def _():
        pltpu.sync_copy(acc_sh.at[pl.ds(row0, 640)],
                        out0_hbm.at[pl.ds(row0, 640)])

    @pl.when(c == 1)
    def _():
        pltpu.sync_copy(acc_sh.at[pl.ds(row0, 640)],
                        out1_hbm.at[pl.ds(row0, 640)])


def _norm_body(degp_ref, dis_ref, inv_ref):
    deg = degp_ref[0] + degp_ref[1] + 1.0
    dis_ref[...] = lax.rsqrt(deg)
    inv_ref[...] = 1.0 / deg


def _tc_norm(deg_partials):
    degp = deg_partials.reshape(2, NPAD // D, D)
    shp = jax.ShapeDtypeStruct((NPAD // D, D), jnp.float32)
    return pl.pallas_call(
        _norm_body, out_shape=(shp, shp))(degp)


def _in_body(x_ref, w_ref, dis_ref, h_ref, g_ref):
    h = jnp.dot(x_ref[...], w_ref[...], preferred_element_type=jnp.float32)
    h_ref[...] = h
    g_ref[...] = h * dis_ref[...]


def _tc_in(x, w, dis):
    shp = jax.ShapeDtypeStruct((N, D), jnp.float32)
    return pl.pallas_call(
        _in_body,
        grid=(N // RB,),
        in_specs=[
            pl.BlockSpec((RB, D), lambda i: (i, 0)),
            pl.BlockSpec((D, D), lambda i: (0, 0)),
            pl.BlockSpec((RB, 1), lambda i: (i, 0)),
        ],
        out_specs=(pl.BlockSpec((RB, D), lambda i: (i, 0)),
                   pl.BlockSpec((RB, D), lambda i: (i, 0))),
        out_shape=(shp, shp))(x, w, dis)


def _mid_body(a0_ref, a1_ref, hp_ref, dis_ref, inv_ref, b_ref, w_ref,
              h_ref, g_ref):
    xn = (a0_ref[...] + a1_ref[...]) * dis_ref[...] \
        + hp_ref[...] * inv_ref[...] + b_ref[...]
    xn = jnp.maximum(xn, 0.0)
    h = jnp.dot(xn, w_ref[...], preferred_element_type=jnp.float32)
    h_ref[...] = h
    g_ref[...] = h * dis_ref[...]


def _tc_mid(a0, a1, h_prev, dis, inv, b, w):
    shp = jax.ShapeDtypeStruct((N, D), jnp.float32)
    return pl.pallas_call(
        _mid_body,
        grid=(N // RB,),
        in_specs=[
            pl.BlockSpec((RB, D), lambda i: (i, 0)),
            pl.BlockSpec((RB, D), lambda i: (i, 0)),
            pl.BlockSpec((RB, D), lambda i: (i, 0)),
            pl.BlockSpec((RB, 1), lambda i: (i, 0)),
            pl.BlockSpec((RB, 1), lambda i: (i, 0)),
            pl.BlockSpec((1, D), lambda i: (0, 0)),
            pl.BlockSpec((D, D), lambda i: (0, 0)),
        ],
        out_specs=(pl.BlockSpec((RB, D), lambda i: (i, 0)),
                   pl.BlockSpec((RB, D), lambda i: (i, 0))),
        out_shape=(shp, shp))(a0, a1, h_prev, dis, inv, b.reshape(1, D), w)


def _out_body(a0_ref, a1_ref, hp_ref, dis_ref, inv_ref, b_ref, o_ref):
    o_ref[...] = (a0_ref[...] + a1_ref[...]) * dis_ref[...] \
        + hp_ref[...] * inv_ref[...] + b_ref[...]


def _tc_out(a0, a1, h_prev, dis, inv, b):
    return pl.pallas_call(
        _out_body,
        grid=(N // RB,),
        in_specs=[
            pl.BlockSpec((RB, D), lambda i: (i, 0)),
            pl.BlockSpec((RB, D), lambda i: (i, 0)),
            pl.BlockSpec((RB, D), lambda i: (i, 0)),
            pl.BlockSpec((RB, 1), lambda i: (i, 0)),
            pl.BlockSpec((RB, 1), lambda i: (i, 0)),
            pl.BlockSpec((1, D), lambda i: (0, 0)),
        ],
        out_specs=pl.BlockSpec((RB, D), lambda i: (i, 0)),
        out_shape=jax.ShapeDtypeStruct((N, D), jnp.float32),
    )(a0, a1, h_prev, dis, inv, b.reshape(1, D))


def kernel(x, edge_index, W1, b1, W2, b2, W3, b3):
    src = edge_index[0].astype(jnp.int32)
    dst = edge_index[1].astype(jnp.int32)

    deg_partials = _sc_degree(dst)
    dis_pad, inv_pad = _tc_norm(deg_partials)
    dis = dis_pad.reshape(NPAD, 1)[:N]
    inv = inv_pad.reshape(NPAD, 1)[:N]

    h1, g1 = _tc_in(x, W1, dis)
    a0, a1 = _sc_aggregate(g1, src, dst)
    h2, g2 = _tc_mid(a0, a1, h1, dis, inv, b1, W2)
    a0, a1 = _sc_aggregate(g2, src, dst)
    h3, g3 = _tc_mid(a0, a1, h2, dis, inv, b2, W3)
    a0, a1 = _sc_aggregate(g3, src, dst)
    return _tc_out(a0, a1, h3, dis, inv, b3)


# speedup vs baseline: 21.7153x; 1.2444x over previous
import functools

import jax
import jax.numpy as jnp
from jax import lax
from jax.experimental import pallas as pl
from jax.experimental.pallas import tpu as pltpu
from jax.experimental.pallas import tpu_sc as plsc

N = 10000
E = 320000
D = 128
NPAD = 10240
EPW = E // 32
CD = 2000
CA = 80
NCH = EPW // CA
RB = 2000

_mesh = functools.partial(
    plsc.VectorSubcoreMesh, core_axis_name="c", subcore_axis_name="s")


@functools.partial(
    pl.kernel,
    out_type=jax.ShapeDtypeStruct((2 * NPAD,), jnp.float32),
    mesh=_mesh(),
    scratch_types=[
        pltpu.VMEM((CD,), jnp.int32),
        pltpu.VMEM((CD,), jnp.float32),
        pltpu.VMEM((640,), jnp.float32),
        pltpu.VMEM_SHARED((NPAD,), jnp.float32),
    ],
)
def _sc_degree(dst_hbm, out_hbm, idx_v, ones_v, zero_v, deg_sh):
    c = lax.axis_index("c")
    s = lax.axis_index("s")
    wid = c * 16 + s

    def fill_ones(i, carry):
        ones_v[pl.ds(i * 16, 16)] = jnp.full((16,), 1.0, jnp.float32)
        return carry

    lax.fori_loop(0, CD // 16, fill_ones, 0)

    def fill_zero(i, carry):
        zero_v[pl.ds(i * 16, 16)] = jnp.zeros((16,), jnp.float32)
        return carry

    lax.fori_loop(0, 640 // 16, fill_zero, 0)

    pltpu.sync_copy(zero_v, deg_sh.at[pl.ds(s * 640, 640)])
    plsc.subcore_barrier()

    def chunk(k, carry):
        base = wid * (E // 32) + k * CD
        pltpu.sync_copy(dst_hbm.at[pl.ds(base, CD)], idx_v)
        pltpu.sync_copy(ones_v, deg_sh.at[idx_v], add=True)
        return carry

    lax.fori_loop(0, (E // 32) // CD, chunk, 0)
    plsc.subcore_barrier()

    pltpu.sync_copy(deg_sh.at[pl.ds(s * 640, 640)],
                    out_hbm.at[pl.ds(c * NPAD + s * 640, 640)])


@functools.partial(
    pl.kernel,
    out_type=jax.ShapeDtypeStruct((2, NPAD, D), jnp.float32),
    mesh=_mesh(),
    scratch_types=[
        pltpu.VMEM((CA,), jnp.int32),
        pltpu.VMEM((CA,), jnp.int32),
        pltpu.VMEM((CA,), jnp.int32),
        pltpu.VMEM((CA,), jnp.int32),
        pltpu.VMEM((CA, D), jnp.float32),
        pltpu.VMEM((CA, D), jnp.float32),
        pltpu.VMEM_SHARED((NPAD, D), jnp.float32),
        pltpu.SemaphoreType.DMA,
        pltpu.SemaphoreType.DMA,
        pltpu.SemaphoreType.DMA,
        pltpu.SemaphoreType.DMA,
    ],
)
def _sc_aggregate(g_hbm, src_hbm, dst_hbm, out_hbm,
                  src0, src1, dst0, dst1, rows0, rows1, acc_sh,
                  sem_g0, sem_g1, sem_i0, sem_i1):
    c = lax.axis_index("c")
    s = lax.axis_index("s")
    srcb = (src0, src1)
    dstb = (dst0, dst1)
    rowsb = (rows0, rows1)
    sem_g = (sem_g0, sem_g1)
    sem_i = (sem_i0, sem_i1)
    wid = c * 16 + s
    ebase = wid * EPW

    def fill_zero(i, carry):
        rows0[i // (D // 16), pl.ds((i % (D // 16)) * 16, 16)] = \
            jnp.zeros((16,), jnp.float32)
        return carry

    lax.fori_loop(0, CA * (D // 16), fill_zero, 0)

    row0 = s * 640
    for off in range(0, 640, CA):
        pltpu.sync_copy(rows0, acc_sh.at[pl.ds(row0 + off, CA)])
    plsc.subcore_barrier()

    pltpu.sync_copy(src_hbm.at[pl.ds(ebase, CA)], src0)
    pltpu.sync_copy(dst_hbm.at[pl.ds(ebase, CA)], dst0)
    pltpu.async_copy(g_hbm.at[src0], rows0, sem_g0)
    pltpu.async_copy(src_hbm.at[pl.ds(ebase + CA, CA)], src1, sem_i1)
    pltpu.async_copy(dst_hbm.at[pl.ds(ebase + CA, CA)], dst1, sem_i1)

    def step(k, b):
        nb = 1 - b
        pltpu.make_async_copy(g_hbm.at[srcb[b]], rowsb[b], sem_g[b]).wait()

        @pl.when(k + 1 < NCH)
        def _():
            pltpu.make_async_copy(
                src_hbm.at[pl.ds(ebase + (k + 1) * CA, CA)],
                srcb[nb], sem_i[nb]).wait()
            pltpu.make_async_copy(
                dst_hbm.at[pl.ds(ebase + (k + 1) * CA, CA)],
                dstb[nb], sem_i[nb]).wait()
            pltpu.async_copy(g_hbm.at[srcb[nb]], rowsb[nb], sem_g[nb])

        pltpu.sync_copy(rowsb[b], acc_sh.at[dstb[b]], add=True)

        @pl.when(k + 2 < NCH)
        def _():
            pltpu.async_copy(
                src_hbm.at[pl.ds(ebase + (k + 2) * CA, CA)],
                srcb[b], sem_i[b])
            pltpu.async_copy(
                dst_hbm.at[pl.ds(ebase + (k + 2) * CA, CA)],
                dstb[b], sem_i[b])

    def pair(gp, carry):
        step(gp * 2, 0)
        step(gp * 2 + 1, 1)
        return carry

    lax.fori_loop(0, NCH // 2, pair, 0)
    if NCH % 2:
        step(NCH - 1, 0)
    plsc.subcore_barrier()

    pltpu.sync_copy(acc_sh.at[pl.ds(row0, 640)],
                    out_hbm.at[c, pl.ds(row0, 640)])


def _norm_body(degp_ref, dis_ref, inv_ref):
    deg = degp_ref[0] + degp_ref[1] + 1.0
    dis_ref[...] = lax.rsqrt(deg)
    inv_ref[...] = 1.0 / deg


def _tc_norm(deg_partials):
    degp = deg_partials.reshape(2, NPAD // D, D)
    shp = jax.ShapeDtypeStruct((NPAD // D, D), jnp.float32)
    return pl.pallas_call(
        _norm_body, out_shape=(shp, shp))(degp)


def _in_body(x_ref, w_ref, dis_ref, h_ref, g_ref):
    h = jnp.dot(x_ref[...], w_ref[...], preferred_element_type=jnp.float32)
    h_ref[...] = h
    g_ref[...] = h * dis_ref[...]


def _tc_in(x, w, dis):
    shp = jax.ShapeDtypeStruct((N, D), jnp.float32)
    return pl.pallas_call(
        _in_body,
        grid=(N // RB,),
        in_specs=[
            pl.BlockSpec((RB, D), lambda i: (i, 0)),
            pl.BlockSpec((D, D), lambda i: (0, 0)),
            pl.BlockSpec((RB, 1), lambda i: (i, 0)),
        ],
        out_specs=(pl.BlockSpec((RB, D), lambda i: (i, 0)),
                   pl.BlockSpec((RB, D), lambda i: (i, 0))),
        out_shape=(shp, shp))(x, w, dis)


def _mid_body(a_ref, hp_ref, dis_ref, inv_ref, b_ref, w_ref, h_ref, g_ref):
    acc = a_ref[0] + a_ref[1]
    xn = acc * dis_ref[...] + hp_ref[...] * inv_ref[...] + b_ref[...]
    xn = jnp.maximum(xn, 0.0)
    h = jnp.dot(xn, w_ref[...], preferred_element_type=jnp.float32)
    h_ref[...] = h
    g_ref[...] = h * dis_ref[...]


def _tc_mid(a, h_prev, dis, inv, b, w):
    shp = jax.ShapeDtypeStruct((N, D), jnp.float32)
    return pl.pallas_call(
        _mid_body,
        grid=(N // RB,),
        in_specs=[
            pl.BlockSpec((2, RB, D), lambda i: (0, i, 0)),
            pl.BlockSpec((RB, D), lambda i: (i, 0)),
            pl.BlockSpec((RB, 1), lambda i: (i, 0)),
            pl.BlockSpec((RB, 1), lambda i: (i, 0)),
            pl.BlockSpec((1, D), lambda i: (0, 0)),
            pl.BlockSpec((D, D), lambda i: (0, 0)),
        ],
        out_specs=(pl.BlockSpec((RB, D), lambda i: (i, 0)),
                   pl.BlockSpec((RB, D), lambda i: (i, 0))),
        out_shape=(shp, shp),
    )(a, h_prev, dis, inv, b.reshape(1, D), w)


def _out_body(a_ref, hp_ref, dis_ref, inv_ref, b_ref, o_ref):
    acc = a_ref[0] + a_ref[1]
    o_ref[...] = acc * dis_ref[...] + hp_ref[...] * inv_ref[...] + b_ref[...]


def _tc_out(a, h_prev, dis, inv, b):
    return pl.pallas_call(
        _out_body,
        grid=(N // RB,),
        in_specs=[
            pl.BlockSpec((2, RB, D), lambda i: (0, i, 0)),
            pl.BlockSpec((RB, D), lambda i: (i, 0)),
            pl.BlockSpec((RB, 1), lambda i: (i, 0)),
            pl.BlockSpec((RB, 1), lambda i: (i, 0)),
            pl.BlockSpec((1, D), lambda i: (0, 0)),
        ],
        out_specs=pl.BlockSpec((RB, D), lambda i: (i, 0)),
        out_shape=jax.ShapeDtypeStruct((N, D), jnp.float32),
    )(a, h_prev, dis, inv, b.reshape(1, D))


def kernel(x, edge_index, W1, b1, W2, b2, W3, b3):
    src = edge_index[0].astype(jnp.int32)
    dst = edge_index[1].astype(jnp.int32)

    deg_partials = _sc_degree(dst)
    dis_pad, inv_pad = _tc_norm(deg_partials)
    dis = dis_pad.reshape(NPAD, 1)[:N]
    inv = inv_pad.reshape(NPAD, 1)[:N]

    h1, g1 = _tc_in(x, W1, dis)
    a = _sc_aggregate(g1, src, dst)
    h2, g2 = _tc_mid(a, h1, dis, inv, b1, W2)
    a = _sc_aggregate(g2, src, dst)
    h3, g3 = _tc_mid(a, h2, dis, inv, b2, W3)
    a = _sc_aggregate(g3, src, dst)
    return _tc_out(a, h3, dis, inv, b3)
